# baseline (speedup 1.0000x reference)
import jax
import jax.numpy as jnp
from jax.experimental import pallas as pl
from jax.experimental.pallas import tpu as pltpu

_D = 1024
_S = 2048


def _body(s_ref, emb_hbm, spe_hbm, pos_hbm, out_hbm,
          spe_v, pos_v, row_v, sem_big, sem_s, sem_p, sem_row):
    big = pltpu.make_async_copy(emb_hbm, out_hbm.at[:, pl.ds(0, _S), :], sem_big)
    big.start()
    b = s_ref[0]
    p = s_ref[1]
    cs = pltpu.make_async_copy(spe_hbm.at[pl.ds(b, 1), :], spe_v, sem_s)
    cs.start()
    cp = pltpu.make_async_copy(pos_hbm.at[pl.ds(p, 1), :], pos_v, sem_p)
    cp.start()
    cs.wait()
    cp.wait()
    row = spe_v[0, :] + pos_v[0, :]
    row_v[...] = jnp.broadcast_to(row[None, None, :], (2, 1, _D))
    cr = pltpu.make_async_copy(row_v, out_hbm.at[:, pl.ds(_S, 1), :], sem_row)
    cr.start()
    big.wait()
    cr.wait()


def kernel(bos_token, embeds, idx, speech_emb, pos_emb):
    s = jnp.concatenate([bos_token.reshape(-1), idx.reshape(-1)]).astype(jnp.int32)
    return pl.pallas_call(
        _body,
        in_specs=[
            pl.BlockSpec(memory_space=pltpu.SMEM),
            pl.BlockSpec(memory_space=pl.ANY),
            pl.BlockSpec(memory_space=pl.ANY),
            pl.BlockSpec(memory_space=pl.ANY),
        ],
        out_specs=pl.BlockSpec(memory_space=pl.ANY),
        out_shape=jax.ShapeDtypeStruct((2, _S + 1, _D), jnp.float32),
        scratch_shapes=[
            pltpu.VMEM((1, _D), jnp.float32),
            pltpu.VMEM((1, _D), jnp.float32),
            pltpu.VMEM((2, 1, _D), jnp.float32),
            pltpu.SemaphoreType.DMA,
            pltpu.SemaphoreType.DMA,
            pltpu.SemaphoreType.DMA,
            pltpu.SemaphoreType.DMA,
        ],
    )(s, embeds, speech_emb, pos_emb)


# speedup vs baseline: 3.4656x; 3.4656x over previous
import jax
import jax.numpy as jnp
from jax.experimental import pallas as pl
from jax.experimental.pallas import tpu as pltpu

_D = 1024
_S = 2048
_BS = 512
_NB = _S // _BS


def _body(s_ref, emb_ref, spe_ref, pos_ref, out_ref):
    i = pl.program_id(0)

    @pl.when(i < _NB)
    def _copy():
        out_ref[...] = emb_ref[...]

    @pl.when(i == _NB)
    def _tail():
        row = spe_ref[0, 0, :] + pos_ref[0, 0, :]
        out_ref[:, 0, :] = jnp.broadcast_to(row[None, :], (2, _D))


def kernel(bos_token, embeds, idx, speech_emb, pos_emb):
    s = jnp.concatenate([bos_token.reshape(-1), idx.reshape(-1)]).astype(jnp.int32)
    spe3 = speech_emb.reshape(speech_emb.shape[0], 1, _D)
    pos3 = pos_emb.reshape(pos_emb.shape[0], 1, _D)
    grid_spec = pltpu.PrefetchScalarGridSpec(
        num_scalar_prefetch=1,
        grid=(_NB + 1,),
        in_specs=[
            pl.BlockSpec((2, _BS, _D), lambda i, s: (0, jnp.minimum(i, _NB - 1), 0)),
            pl.BlockSpec((1, 1, _D), lambda i, s: (s[0], 0, 0)),
            pl.BlockSpec((1, 1, _D), lambda i, s: (s[1], 0, 0)),
        ],
        out_specs=pl.BlockSpec((2, _BS, _D), lambda i, s: (0, i, 0)),
    )
    return pl.pallas_call(
        _body,
        grid_spec=grid_spec,
        out_shape=jax.ShapeDtypeStruct((2, _S + 1, _D), jnp.float32),
    )(s, embeds, spe3, pos3)


# speedup vs baseline: 7.1166x; 2.0535x over previous
import functools

import jax
import jax.numpy as jnp
from jax import lax
from jax.experimental import pallas as pl
from jax.experimental.pallas import tpu as pltpu
from jax.experimental.pallas import tpu_sc as plsc

_D = 1024
_S = 2048
_NCORE = 2
_NSUB = 16
_RPW = _S // _NSUB
_CH = 32
_NCH = _RPW // _CH

_mesh = plsc.VectorSubcoreMesh(
    core_axis_name="c", subcore_axis_name="s",
    num_cores=_NCORE, num_subcores=_NSUB,
)


@functools.partial(
    pl.kernel,
    out_type=jax.ShapeDtypeStruct((2, _S + 1, _D), jnp.float32),
    mesh=_mesh,
    scratch_types=[
        pltpu.VMEM((_CH, _D), jnp.float32),
        pltpu.VMEM((_CH, _D), jnp.float32),
        pltpu.SemaphoreType.DMA,
        pltpu.SemaphoreType.DMA,
        pltpu.SemaphoreType.DMA,
        pltpu.SemaphoreType.DMA,
    ],
)
def _sc_copy(emb_hbm, out_hbm, bufa, bufb, sem_ia, sem_ib, sem_oa, sem_ob):
    c = lax.axis_index("c")
    s = lax.axis_index("s")
    r0 = s * _RPW

    bufs = (bufa, bufb)
    sem_in = (sem_ia, sem_ib)
    sem_out = (sem_oa, sem_ob)

    def run(cc):
        def mk_in(t):
            return pltpu.make_async_copy(
                emb_hbm.at[cc, pl.ds(r0 + t * _CH, _CH), :],
                bufs[t % 2], sem_in[t % 2])

        def mk_out(t):
            return pltpu.make_async_copy(
                bufs[t % 2], out_hbm.at[cc, pl.ds(r0 + t * _CH, _CH), :],
                sem_out[t % 2])

        mk_in(0).start()
        mk_in(1).start()
        for t in range(_NCH):
            mk_in(t).wait()
            mk_out(t).start()
            mk_out(t).wait()
            if t + 2 < _NCH:
                mk_in(t + 2).start()

    @pl.when(c == 0)
    def _c0():
        run(0)

    @pl.when(c == 1)
    def _c1():
        run(1)


def _tc_row_body(s_ref, out_in_hbm, spe_hbm, pos_hbm, out_hbm,
                 spe_v, pos_v, row_v, sem_s, sem_p, sem_row):
    b = s_ref[0]
    p = s_ref[1]
    cs = pltpu.make_async_copy(spe_hbm.at[pl.ds(b, 1), :], spe_v, sem_s)
    cs.start()
    cp = pltpu.make_async_copy(pos_hbm.at[pl.ds(p, 1), :], pos_v, sem_p)
    cp.start()
    cs.wait()
    cp.wait()
    row = spe_v[0, :] + pos_v[0, :]
    row_v[...] = jnp.broadcast_to(row[None, None, :], (2, 1, _D))
    cr = pltpu.make_async_copy(row_v, out_hbm.at[:, pl.ds(_S, 1), :], sem_row)
    cr.start()
    cr.wait()


def kernel(bos_token, embeds, idx, speech_emb, pos_emb):
    partial = _sc_copy(embeds)
    s = jnp.concatenate([bos_token.reshape(-1), idx.reshape(-1)]).astype(jnp.int32)
    return pl.pallas_call(
        _tc_row_body,
        in_specs=[
            pl.BlockSpec(memory_space=pltpu.SMEM),
            pl.BlockSpec(memory_space=pl.ANY),
            pl.BlockSpec(memory_space=pl.ANY),
            pl.BlockSpec(memory_space=pl.ANY),
        ],
        out_specs=pl.BlockSpec(memory_space=pl.ANY),
        out_shape=jax.ShapeDtypeStruct((2, _S + 1, _D), jnp.float32),
        input_output_aliases={1: 0},
        scratch_shapes=[
            pltpu.VMEM((1, _D), jnp.float32),
            pltpu.VMEM((1, _D), jnp.float32),
            pltpu.VMEM((2, 1, _D), jnp.float32),
            pltpu.SemaphoreType.DMA,
            pltpu.SemaphoreType.DMA,
            pltpu.SemaphoreType.DMA,
        ],
    )(s, partial, speech_emb, pos_emb)
